# Initial kernel scaffold; baseline (speedup 1.0000x reference)
import jax
import jax.numpy as jnp
from jax.experimental import pallas as pl


def kernel(node_features, senders, receivers, valid, W_e1, b_e1, W_n1, b_n1, W_e2, b_e2, W_n2, b_n2, W_e3, b_e3, W_n3, b_n3):
    raise NotImplementedError("write your pallas kernel here")


# speedup vs baseline: 1.5680x; 1.5680x over previous
import functools

import jax
import jax.numpy as jnp
from jax import lax
from jax.experimental import pallas as pl
from jax.experimental.pallas import tpu as pltpu
from jax.experimental.pallas import tpu_sc as plsc

N = 32768
E = 294912
D = 256
H = 256

NC = 2
NS = 16
LANES = 16
NW = NC * NS

CBC = 128
EPW = E // NW
NB = EPW // CBC

BN = 256
NBLK = N // BN
CHK = 512


def _mm(A, W, b, relu=False, A2=None, W2=None, bm=512):
    M, K = A.shape
    Nc = W.shape[1]

    if A2 is None:
        def body(a_ref, w_ref, b_ref, o_ref):
            acc = jnp.dot(a_ref[...], w_ref[...],
                          preferred_element_type=jnp.float32)
            acc = acc + b_ref[...]
            if relu:
                acc = jnp.maximum(acc, 0.0)
            o_ref[...] = acc
        in_specs = [
            pl.BlockSpec((bm, K), lambda m: (m, 0)),
            pl.BlockSpec((K, Nc), lambda m: (0, 0)),
            pl.BlockSpec((1, Nc), lambda m: (0, 0)),
        ]
        args = (A, W, b.reshape(1, Nc))
    else:
        K2 = A2.shape[1]

        def body(a_ref, w_ref, a2_ref, w2_ref, b_ref, o_ref):
            acc = jnp.dot(a_ref[...], w_ref[...],
                          preferred_element_type=jnp.float32)
            acc = acc + jnp.dot(a2_ref[...], w2_ref[...],
                                preferred_element_type=jnp.float32)
            acc = acc + b_ref[...]
            if relu:
                acc = jnp.maximum(acc, 0.0)
            o_ref[...] = acc
        in_specs = [
            pl.BlockSpec((bm, K), lambda m: (m, 0)),
            pl.BlockSpec((K, Nc), lambda m: (0, 0)),
            pl.BlockSpec((bm, K2), lambda m: (m, 0)),
            pl.BlockSpec((K2, Nc), lambda m: (0, 0)),
            pl.BlockSpec((1, Nc), lambda m: (0, 0)),
        ]
        args = (A, W, A2, W2, b.reshape(1, Nc))

    return pl.pallas_call(
        body,
        grid=(M // bm,),
        in_specs=in_specs,
        out_specs=pl.BlockSpec((bm, Nc), lambda m: (m, 0)),
        out_shape=jax.ShapeDtypeStruct((M, Nc), jnp.float32),
    )(*args)


def _mm2(A, W1, W2, bm=512):
    M, K = A.shape
    Nc = W1.shape[1]

    def body(a_ref, w1_ref, w2_ref, o1_ref, o2_ref):
        a = a_ref[...]
        o1_ref[...] = jnp.dot(a, w1_ref[...], preferred_element_type=jnp.float32)
        o2_ref[...] = jnp.dot(a, w2_ref[...], preferred_element_type=jnp.float32)

    return pl.pallas_call(
        body,
        grid=(M // bm,),
        in_specs=[
            pl.BlockSpec((bm, K), lambda m: (m, 0)),
            pl.BlockSpec((K, Nc), lambda m: (0, 0)),
            pl.BlockSpec((K, Nc), lambda m: (0, 0)),
        ],
        out_specs=[
            pl.BlockSpec((bm, Nc), lambda m: (m, 0)),
            pl.BlockSpec((bm, Nc), lambda m: (m, 0)),
        ],
        out_shape=[
            jax.ShapeDtypeStruct((M, Nc), jnp.float32),
            jax.ShapeDtypeStruct((M, Nc), jnp.float32),
        ],
    )(A, W1, W2)


def _edge_bias(valid_col, c1, be, bm=1024):
    def body(v_ref, c_ref, b_ref, o_ref):
        o_ref[...] = v_ref[...] * c_ref[...] + b_ref[...]

    return pl.pallas_call(
        body,
        grid=(E // bm,),
        in_specs=[
            pl.BlockSpec((bm, 1), lambda m: (m, 0)),
            pl.BlockSpec((1, H), lambda m: (0, 0)),
            pl.BlockSpec((1, H), lambda m: (0, 0)),
        ],
        out_specs=pl.BlockSpec((bm, H), lambda m: (m, 0)),
        out_shape=jax.ShapeDtypeStruct((E, H), jnp.float32),
    )(valid_col, c1.reshape(1, H), be.reshape(1, H))


def _segsum(vals, ridx2d, offs):
    def body(offs_ref, vals_ref, ridx_ref, o_ref, vbuf, ibuf, sem_v, sem_i):
        b = pl.program_id(0)
        off0 = offs_ref[b]
        off1 = offs_ref[b + 1]
        base = (off0 // 8) * 8
        nch = lax.div(off1 - base + (CHK - 1), CHK)
        nstart = b * BN
        o_ref[...] = jnp.zeros((BN, H), jnp.float32)

        def chunk(t, c):
            cur = base + t * CHK
            cl = jnp.minimum(cur, E - CHK)
            cpv = pltpu.make_async_copy(vals_ref.at[pl.ds(cl, CHK)], vbuf, sem_v)
            cpi = pltpu.make_async_copy(ridx_ref.at[pl.ds(cl, CHK)], ibuf, sem_i)
            cpv.start()
            cpi.start()
            cpv.wait()
            cpi.wait()
            p_row = cl + lax.broadcasted_iota(jnp.int32, (CHK, 1), 0)
            ok = (p_row >= jnp.maximum(cur, off0)) & (p_row < off1)
            lr = ibuf[...] - nstart
            hit = (lr == lax.broadcasted_iota(jnp.int32, (CHK, BN), 1)) & ok
            onehot = jnp.where(hit, 1.0, 0.0)
            part = lax.dot_general(onehot, vbuf[...],
                                   dimension_numbers=(((0,), (0,)), ((), ())),
                                   preferred_element_type=jnp.float32)
            o_ref[...] = o_ref[...] + part
            return c

        lax.fori_loop(0, nch, chunk, 0)

    return pl.pallas_call(
        body,
        grid=(NBLK,),
        in_specs=[
            pl.BlockSpec(memory_space=pltpu.MemorySpace.SMEM),
            pl.BlockSpec(memory_space=pltpu.MemorySpace.HBM),
            pl.BlockSpec(memory_space=pltpu.MemorySpace.HBM),
        ],
        out_specs=pl.BlockSpec((BN, H), lambda b: (b, 0)),
        out_shape=jax.ShapeDtypeStruct((N, H), jnp.float32),
        scratch_shapes=[
            pltpu.VMEM((CHK, H), jnp.float32),
            pltpu.VMEM((CHK, 1), jnp.int32),
            pltpu.SemaphoreType.DMA,
            pltpu.SemaphoreType.DMA,
        ],
    )(offs, vals, ridx2d)


def _make_combine(relu):
    mesh = plsc.VectorSubcoreMesh(core_axis_name="c", subcore_axis_name="s",
                                  num_cores=NC, num_subcores=NS)

    @functools.partial(
        pl.kernel,
        mesh=mesh,
        out_type=jax.ShapeDtypeStruct((E, H), jnp.float32),
        scratch_types=[
            pltpu.VMEM((CBC,), jnp.int32),
            pltpu.VMEM((CBC,), jnp.int32),
            pltpu.VMEM((CBC, H), jnp.float32),
            pltpu.VMEM((CBC, H), jnp.float32),
            pltpu.VMEM((CBC, H), jnp.float32),
            pltpu.SemaphoreType.DMA,
            pltpu.SemaphoreType.DMA,
        ],
    )
    def combine(ps_hbm, pr_hbm, e1_hbm, idxs_hbm, idxr_hbm, out_hbm,
                idxs_v, idxr_v, bufa, bufb, bufc, sema, semb):
        cid = lax.axis_index("c")
        sid = lax.axis_index("s")
        wid = sid * NC + cid

        def batch(j, carry):
            e0 = wid * EPW + j * CBC
            pltpu.sync_copy(idxs_hbm.at[pl.ds(e0, CBC)], idxs_v)
            pltpu.sync_copy(idxr_hbm.at[pl.ds(e0, CBC)], idxr_v)
            cpa = pltpu.async_copy(ps_hbm.at[idxs_v], bufa, sema)
            cpb = pltpu.async_copy(pr_hbm.at[idxr_v], bufb, semb)
            pltpu.sync_copy(e1_hbm.at[pl.ds(e0, CBC)], bufc)
            cpa.wait()
            cpb.wait()

            def row(r, rc):
                for kk in range(H // LANES):
                    sl = pl.ds(kk * LANES, LANES)
                    v = bufa[r, sl] + bufb[r, sl] + bufc[r, sl]
                    if relu:
                        v = jnp.maximum(v, 0.0)
                    bufc[r, sl] = v
                return rc

            lax.fori_loop(0, CBC, row, 0)
            pltpu.sync_copy(bufc, out_hbm.at[pl.ds(e0, CBC)])
            return carry

        lax.fori_loop(0, NB, batch, 0)

    return combine


_combine_relu = _make_combine(True)
_combine_lin = _make_combine(False)


def kernel(node_features, senders, receivers, valid,
           W_e1, b_e1, W_n1, b_n1,
           W_e2, b_e2, W_n2, b_n2,
           W_e3, b_e3, W_n3, b_n3):
    perm = jnp.argsort(receivers)
    s_s = senders[perm]
    r_s = receivers[perm]
    valid_col = valid[perm].astype(jnp.float32).reshape(E, 1)
    offs = jnp.searchsorted(r_s, jnp.arange(0, N + 1, BN, dtype=jnp.int32)
                            ).astype(jnp.int32)
    r2d = r_s.reshape(E, 1)

    def layer(nodes, e_prev, We, be, Wn, bn, relu, first):
        K0 = We.shape[0] // 3
        Wee, Wes, Wer = We[:K0], We[K0:2 * K0], We[2 * K0:]
        Ps, Pr = _mm2(nodes, Wes, Wer)
        if first:
            e_term = _edge_bias(valid_col, jnp.sum(Wee, axis=0), be)
        else:
            e_term = _mm(e_prev, Wee, be)
        comb = _combine_relu if relu else _combine_lin
        e_new = comb(Ps, Pr, e_term, s_s, r_s)
        recv = _segsum(e_new, r2d, offs)
        Kn = nodes.shape[1]
        n_new = _mm(nodes, Wn[:Kn], bn, relu=relu, A2=recv, W2=Wn[Kn:])
        return n_new, e_new

    n1, e1 = layer(node_features, None, W_e1, b_e1, W_n1, b_n1, True, True)
    n2, e2 = layer(n1, e1, W_e2, b_e2, W_n2, b_n2, True, False)
    n3, _ = layer(n2, e2, W_e3, b_e3, W_n3, b_n3, False, False)
    return n3


# speedup vs baseline: 2.1191x; 1.3514x over previous
import functools

import jax
import jax.numpy as jnp
from jax import lax
from jax.experimental import pallas as pl
from jax.experimental.pallas import tpu as pltpu
from jax.experimental.pallas import tpu_sc as plsc

N = 32768
E = 294912
D = 256
H = 256

NC = 2
NS = 16
LANES = 16
NW = NC * NS

CBC = 48
EPW = E // NW
NB = EPW // CBC

BN = 256
NBLK = N // BN
CHK = 512


def _mm(A, W, b, relu=False, A2=None, W2=None, bm=512):
    M, K = A.shape
    Nc = W.shape[1]

    if A2 is None:
        def body(a_ref, w_ref, b_ref, o_ref):
            acc = jnp.dot(a_ref[...], w_ref[...],
                          preferred_element_type=jnp.float32)
            acc = acc + b_ref[...]
            if relu:
                acc = jnp.maximum(acc, 0.0)
            o_ref[...] = acc
        in_specs = [
            pl.BlockSpec((bm, K), lambda m: (m, 0)),
            pl.BlockSpec((K, Nc), lambda m: (0, 0)),
            pl.BlockSpec((1, Nc), lambda m: (0, 0)),
        ]
        args = (A, W, b.reshape(1, Nc))
    else:
        K2 = A2.shape[1]

        def body(a_ref, w_ref, a2_ref, w2_ref, b_ref, o_ref):
            acc = jnp.dot(a_ref[...], w_ref[...],
                          preferred_element_type=jnp.float32)
            acc = acc + jnp.dot(a2_ref[...], w2_ref[...],
                                preferred_element_type=jnp.float32)
            acc = acc + b_ref[...]
            if relu:
                acc = jnp.maximum(acc, 0.0)
            o_ref[...] = acc
        in_specs = [
            pl.BlockSpec((bm, K), lambda m: (m, 0)),
            pl.BlockSpec((K, Nc), lambda m: (0, 0)),
            pl.BlockSpec((bm, K2), lambda m: (m, 0)),
            pl.BlockSpec((K2, Nc), lambda m: (0, 0)),
            pl.BlockSpec((1, Nc), lambda m: (0, 0)),
        ]
        args = (A, W, A2, W2, b.reshape(1, Nc))

    return pl.pallas_call(
        body,
        grid=(M // bm,),
        in_specs=in_specs,
        out_specs=pl.BlockSpec((bm, Nc), lambda m: (m, 0)),
        out_shape=jax.ShapeDtypeStruct((M, Nc), jnp.float32),
    )(*args)


def _mm2(A, W1, W2, bm=512):
    M, K = A.shape
    Nc = W1.shape[1]

    def body(a_ref, w1_ref, w2_ref, o1_ref, o2_ref):
        a = a_ref[...]
        o1_ref[...] = jnp.dot(a, w1_ref[...], preferred_element_type=jnp.float32)
        o2_ref[...] = jnp.dot(a, w2_ref[...], preferred_element_type=jnp.float32)

    return pl.pallas_call(
        body,
        grid=(M // bm,),
        in_specs=[
            pl.BlockSpec((bm, K), lambda m: (m, 0)),
            pl.BlockSpec((K, Nc), lambda m: (0, 0)),
            pl.BlockSpec((K, Nc), lambda m: (0, 0)),
        ],
        out_specs=[
            pl.BlockSpec((bm, Nc), lambda m: (m, 0)),
            pl.BlockSpec((bm, Nc), lambda m: (m, 0)),
        ],
        out_shape=[
            jax.ShapeDtypeStruct((M, Nc), jnp.float32),
            jax.ShapeDtypeStruct((M, Nc), jnp.float32),
        ],
    )(A, W1, W2)


def _edge_bias(valid_col, c1, be, bm=1024):
    def body(v_ref, c_ref, b_ref, o_ref):
        o_ref[...] = v_ref[...] * c_ref[...] + b_ref[...]

    return pl.pallas_call(
        body,
        grid=(E // bm,),
        in_specs=[
            pl.BlockSpec((bm, 1), lambda m: (m, 0)),
            pl.BlockSpec((1, H), lambda m: (0, 0)),
            pl.BlockSpec((1, H), lambda m: (0, 0)),
        ],
        out_specs=pl.BlockSpec((bm, H), lambda m: (m, 0)),
        out_shape=jax.ShapeDtypeStruct((E, H), jnp.float32),
    )(valid_col, c1.reshape(1, H), be.reshape(1, H))


def _segsum(vals, ridx2d, offs):
    def body(offs_ref, vals_ref, ridx_ref, o_ref,
             vbuf0, vbuf1, ibuf0, ibuf1, sv0, sv1, si0, si1):
        b = pl.program_id(0)
        off0 = offs_ref[b]
        off1 = offs_ref[b + 1]
        base = (off0 // 8) * 8
        nch = lax.div(off1 - base + (CHK - 1), CHK)
        nstart = b * BN
        o_ref[...] = jnp.zeros((BN, H), jnp.float32)
        vb = (vbuf0, vbuf1)
        ib = (ibuf0, ibuf1)
        sv = (sv0, sv1)
        si = (si0, si1)

        def clamp(t):
            return jnp.minimum(base + t * CHK, E - CHK)

        def start(t, s):
            cl = clamp(t)
            pltpu.make_async_copy(vals_ref.at[pl.ds(cl, CHK)], vb[s], sv[s]
                                  ).start()
            pltpu.make_async_copy(ridx_ref.at[pl.ds(cl, CHK)], ib[s], si[s]
                                  ).start()

        @pl.when(nch > 0)
        def _go():
            start(0, 0)

            def chunk_s(t, s):
                @pl.when(t + 1 < nch)
                def _pre():
                    start(t + 1, 1 - s)

                pltpu.make_async_copy(vals_ref.at[pl.ds(0, CHK)], vb[s], sv[s]
                                      ).wait()
                pltpu.make_async_copy(ridx_ref.at[pl.ds(0, CHK)], ib[s], si[s]
                                      ).wait()
                cur = base + t * CHK
                cl = clamp(t)
                p_row = cl + lax.broadcasted_iota(jnp.int32, (CHK, 1), 0)
                ok = (p_row >= jnp.maximum(cur, off0)) & (p_row < off1)
                lr = ib[s][...] - nstart
                hit = (lr == lax.broadcasted_iota(jnp.int32, (CHK, BN), 1)) & ok
                onehot = jnp.where(hit, 1.0, 0.0)
                part = lax.dot_general(onehot, vb[s][...],
                                       dimension_numbers=(((0,), (0,)), ((), ())),
                                       preferred_element_type=jnp.float32)
                o_ref[...] = o_ref[...] + part

            def pair(tt, c):
                @pl.when(2 * tt < nch)
                def _a():
                    chunk_s(2 * tt, 0)

                @pl.when(2 * tt + 1 < nch)
                def _b():
                    chunk_s(2 * tt + 1, 1)

                return c

            lax.fori_loop(0, lax.div(nch + 1, 2), pair, 0)

    return pl.pallas_call(
        body,
        grid=(NBLK,),
        in_specs=[
            pl.BlockSpec(memory_space=pltpu.MemorySpace.SMEM),
            pl.BlockSpec(memory_space=pltpu.MemorySpace.HBM),
            pl.BlockSpec(memory_space=pltpu.MemorySpace.HBM),
        ],
        out_specs=pl.BlockSpec((BN, H), lambda b: (b, 0)),
        out_shape=jax.ShapeDtypeStruct((N, H), jnp.float32),
        scratch_shapes=[
            pltpu.VMEM((CHK, H), jnp.float32),
            pltpu.VMEM((CHK, H), jnp.float32),
            pltpu.VMEM((CHK, 1), jnp.int32),
            pltpu.VMEM((CHK, 1), jnp.int32),
            pltpu.SemaphoreType.DMA,
            pltpu.SemaphoreType.DMA,
            pltpu.SemaphoreType.DMA,
            pltpu.SemaphoreType.DMA,
        ],
    )(offs, vals, ridx2d)


def _make_combine(relu):
    mesh = plsc.VectorSubcoreMesh(core_axis_name="c", subcore_axis_name="s",
                                  num_cores=NC, num_subcores=NS)

    scratch = [pltpu.VMEM((EPW,), jnp.int32), pltpu.VMEM((EPW,), jnp.int32)]
    for _ in range(2):
        scratch += [pltpu.VMEM((CBC, H), jnp.float32)] * 4
    scratch += [pltpu.SemaphoreType.DMA] * 8

    @functools.partial(
        pl.kernel,
        mesh=mesh,
        out_type=jax.ShapeDtypeStruct((E, H), jnp.float32),
        scratch_types=scratch,
    )
    def combine(ps_hbm, pr_hbm, e1_hbm, idxs_hbm, idxr_hbm, out_hbm, *bufs):
        idxs_v, idxr_v = bufs[0], bufs[1]
        ba = (bufs[2], bufs[6])
        bb = (bufs[3], bufs[7])
        bc = (bufs[4], bufs[8])
        bo = (bufs[5], bufs[9])
        sa = (bufs[10], bufs[11])
        sb = (bufs[12], bufs[13])
        sc = (bufs[14], bufs[15])
        so = (bufs[16], bufs[17])

        cid = lax.axis_index("c")
        sid = lax.axis_index("s")
        wid = sid * NC + cid
        g0 = wid * EPW
        pltpu.sync_copy(idxs_hbm.at[pl.ds(g0, EPW)], idxs_v)
        pltpu.sync_copy(idxr_hbm.at[pl.ds(g0, EPW)], idxr_v)

        def start_reads(j, s):
            pltpu.async_copy(ps_hbm.at[idxs_v.at[pl.ds(j * CBC, CBC)]],
                             ba[s], sa[s])
            pltpu.async_copy(pr_hbm.at[idxr_v.at[pl.ds(j * CBC, CBC)]],
                             bb[s], sb[s])
            pltpu.async_copy(e1_hbm.at[pl.ds(g0 + j * CBC, CBC)], bc[s], sc[s])

        def process(j, s):
            pltpu.make_async_copy(ps_hbm.at[pl.ds(0, CBC)], ba[s], sa[s]).wait()
            pltpu.make_async_copy(pr_hbm.at[pl.ds(0, CBC)], bb[s], sb[s]).wait()
            pltpu.make_async_copy(e1_hbm.at[pl.ds(0, CBC)], bc[s], sc[s]).wait()

            @pl.when(j >= 2)
            def _wait_prev_write():
                pltpu.make_async_copy(bo[s], out_hbm.at[pl.ds(0, CBC)],
                                      so[s]).wait()

            def row(r, rc):
                for kk in range(H // LANES):
                    sl = pl.ds(kk * LANES, LANES)
                    v = ba[s][r, sl] + bb[s][r, sl] + bc[s][r, sl]
                    if relu:
                        v = jnp.maximum(v, 0.0)
                    bo[s][r, sl] = v
                return rc

            lax.fori_loop(0, CBC, row, 0)
            pltpu.async_copy(bo[s], out_hbm.at[pl.ds(g0 + j * CBC, CBC)], so[s])

            @pl.when(j + 2 < NB)
            def _prefetch():
                start_reads(j + 2, s)

        start_reads(0, 0)
        start_reads(1, 1)

        def pair(jj, carry):
            process(2 * jj, 0)
            process(2 * jj + 1, 1)
            return carry

        lax.fori_loop(0, NB // 2, pair, 0)
        pltpu.make_async_copy(bo[0], out_hbm.at[pl.ds(0, CBC)], so[0]).wait()
        pltpu.make_async_copy(bo[1], out_hbm.at[pl.ds(0, CBC)], so[1]).wait()

    return combine


_combine_relu = _make_combine(True)
_combine_lin = _make_combine(False)


def kernel(node_features, senders, receivers, valid,
           W_e1, b_e1, W_n1, b_n1,
           W_e2, b_e2, W_n2, b_n2,
           W_e3, b_e3, W_n3, b_n3):
    perm = jnp.argsort(receivers)
    s_s = senders[perm]
    r_s = receivers[perm]
    valid_col = valid[perm].astype(jnp.float32).reshape(E, 1)
    offs = jnp.searchsorted(r_s, jnp.arange(0, N + 1, BN, dtype=jnp.int32)
                            ).astype(jnp.int32)
    r2d = r_s.reshape(E, 1)

    def layer(nodes, e_prev, We, be, Wn, bn, relu, first):
        K0 = We.shape[0] // 3
        Wee, Wes, Wer = We[:K0], We[K0:2 * K0], We[2 * K0:]
        Ps, Pr = _mm2(nodes, Wes, Wer)
        if first:
            e_term = _edge_bias(valid_col, jnp.sum(Wee, axis=0), be)
        else:
            e_term = _mm(e_prev, Wee, be)
        comb = _combine_relu if relu else _combine_lin
        e_new = comb(Ps, Pr, e_term, s_s, r_s)
        recv = _segsum(e_new, r2d, offs)
        Kn = nodes.shape[1]
        n_new = _mm(nodes, Wn[:Kn], bn, relu=relu, A2=recv, W2=Wn[Kn:])
        return n_new, e_new

    n1, e1 = layer(node_features, None, W_e1, b_e1, W_n1, b_n1, True, True)
    n2, e2 = layer(n1, e1, W_e2, b_e2, W_n2, b_n2, True, False)
    n3, _ = layer(n2, e2, W_e3, b_e3, W_n3, b_n3, False, False)
    return n3


# speedup vs baseline: 2.2694x; 1.0709x over previous
import functools

import jax
import jax.numpy as jnp
from jax import lax
from jax.experimental import pallas as pl
from jax.experimental.pallas import tpu as pltpu
from jax.experimental.pallas import tpu_sc as plsc

N = 32768
E = 294912
D = 256
H = 256

NC = 2
NS = 16
LANES = 16
NW = NC * NS

CBC = 48
EPW = E // NW
NB = EPW // CBC

BN = 256
NBLK = N // BN
CHK = 1024


def _mm(A, W, b, relu=False, A2=None, W2=None, bm=512):
    M, K = A.shape
    Nc = W.shape[1]

    if A2 is None:
        def body(a_ref, w_ref, b_ref, o_ref):
            acc = jnp.dot(a_ref[...], w_ref[...],
                          preferred_element_type=jnp.float32)
            acc = acc + b_ref[...]
            if relu:
                acc = jnp.maximum(acc, 0.0)
            o_ref[...] = acc
        in_specs = [
            pl.BlockSpec((bm, K), lambda m: (m, 0)),
            pl.BlockSpec((K, Nc), lambda m: (0, 0)),
            pl.BlockSpec((1, Nc), lambda m: (0, 0)),
        ]
        args = (A, W, b.reshape(1, Nc))
    else:
        K2 = A2.shape[1]

        def body(a_ref, w_ref, a2_ref, w2_ref, b_ref, o_ref):
            acc = jnp.dot(a_ref[...], w_ref[...],
                          preferred_element_type=jnp.float32)
            acc = acc + jnp.dot(a2_ref[...], w2_ref[...],
                                preferred_element_type=jnp.float32)
            acc = acc + b_ref[...]
            if relu:
                acc = jnp.maximum(acc, 0.0)
            o_ref[...] = acc
        in_specs = [
            pl.BlockSpec((bm, K), lambda m: (m, 0)),
            pl.BlockSpec((K, Nc), lambda m: (0, 0)),
            pl.BlockSpec((bm, K2), lambda m: (m, 0)),
            pl.BlockSpec((K2, Nc), lambda m: (0, 0)),
            pl.BlockSpec((1, Nc), lambda m: (0, 0)),
        ]
        args = (A, W, A2, W2, b.reshape(1, Nc))

    return pl.pallas_call(
        body,
        grid=(M // bm,),
        in_specs=in_specs,
        out_specs=pl.BlockSpec((bm, Nc), lambda m: (m, 0)),
        out_shape=jax.ShapeDtypeStruct((M, Nc), jnp.float32),
    )(*args)


def _mm_node(A, W1, recv, W2, b, relu, Wp1=None, Wp2=None, bm=512):
    M, K = A.shape
    Nc = W1.shape[1]
    proj = Wp1 is not None

    def body(*refs):
        if proj:
            (a_ref, w1_ref, r_ref, w2_ref, b_ref, p1_ref, p2_ref,
             on_ref, o1_ref, o2_ref) = refs
        else:
            a_ref, w1_ref, r_ref, w2_ref, b_ref, on_ref = refs
        acc = jnp.dot(a_ref[...], w1_ref[...], preferred_element_type=jnp.float32)
        acc = acc + jnp.dot(r_ref[...], w2_ref[...],
                            preferred_element_type=jnp.float32)
        acc = acc + b_ref[...]
        if relu:
            acc = jnp.maximum(acc, 0.0)
        on_ref[...] = acc
        if proj:
            o1_ref[...] = jnp.dot(acc, p1_ref[...],
                                  preferred_element_type=jnp.float32)
            o2_ref[...] = jnp.dot(acc, p2_ref[...],
                                  preferred_element_type=jnp.float32)

    in_specs = [
        pl.BlockSpec((bm, K), lambda m: (m, 0)),
        pl.BlockSpec((K, Nc), lambda m: (0, 0)),
        pl.BlockSpec((bm, Nc), lambda m: (m, 0)),
        pl.BlockSpec((Nc, Nc), lambda m: (0, 0)),
        pl.BlockSpec((1, Nc), lambda m: (0, 0)),
    ]
    args = [A, W1, recv, W2, b.reshape(1, Nc)]
    out_specs = [pl.BlockSpec((bm, Nc), lambda m: (m, 0))]
    out_shape = [jax.ShapeDtypeStruct((M, Nc), jnp.float32)]
    if proj:
        in_specs += [pl.BlockSpec((Nc, Nc), lambda m: (0, 0))] * 2
        args += [Wp1, Wp2]
        out_specs += [pl.BlockSpec((bm, Nc), lambda m: (m, 0))] * 2
        out_shape += [jax.ShapeDtypeStruct((M, Nc), jnp.float32)] * 2

    res = pl.pallas_call(
        body,
        grid=(M // bm,),
        in_specs=in_specs,
        out_specs=out_specs,
        out_shape=out_shape,
    )(*args)
    return res if proj else res[0]


def _mm2(A, W1, W2, bm=512):
    M, K = A.shape
    Nc = W1.shape[1]

    def body(a_ref, w1_ref, w2_ref, o1_ref, o2_ref):
        a = a_ref[...]
        o1_ref[...] = jnp.dot(a, w1_ref[...], preferred_element_type=jnp.float32)
        o2_ref[...] = jnp.dot(a, w2_ref[...], preferred_element_type=jnp.float32)

    return pl.pallas_call(
        body,
        grid=(M // bm,),
        in_specs=[
            pl.BlockSpec((bm, K), lambda m: (m, 0)),
            pl.BlockSpec((K, Nc), lambda m: (0, 0)),
            pl.BlockSpec((K, Nc), lambda m: (0, 0)),
        ],
        out_specs=[
            pl.BlockSpec((bm, Nc), lambda m: (m, 0)),
            pl.BlockSpec((bm, Nc), lambda m: (m, 0)),
        ],
        out_shape=[
            jax.ShapeDtypeStruct((M, Nc), jnp.float32),
            jax.ShapeDtypeStruct((M, Nc), jnp.float32),
        ],
    )(A, W1, W2)


def _edge_bias(valid_col, c1, be, bm=1024):
    def body(v_ref, c_ref, b_ref, o_ref):
        o_ref[...] = v_ref[...] * c_ref[...] + b_ref[...]

    return pl.pallas_call(
        body,
        grid=(E // bm,),
        in_specs=[
            pl.BlockSpec((bm, 1), lambda m: (m, 0)),
            pl.BlockSpec((1, H), lambda m: (0, 0)),
            pl.BlockSpec((1, H), lambda m: (0, 0)),
        ],
        out_specs=pl.BlockSpec((bm, H), lambda m: (m, 0)),
        out_shape=jax.ShapeDtypeStruct((E, H), jnp.float32),
    )(valid_col, c1.reshape(1, H), be.reshape(1, H))


def _segsum(vals, ridx2d, offs):
    def body(offs_ref, vals_ref, ridx_ref, o_ref,
             vbuf0, vbuf1, ibuf0, ibuf1, sv0, sv1, si0, si1):
        b = pl.program_id(0)
        off0 = offs_ref[b]
        off1 = offs_ref[b + 1]
        base = (off0 // 8) * 8
        nch = lax.div(off1 - base + (CHK - 1), CHK)
        nstart = b * BN
        o_ref[...] = jnp.zeros((BN, H), jnp.float32)
        vb = (vbuf0, vbuf1)
        ib = (ibuf0, ibuf1)
        sv = (sv0, sv1)
        si = (si0, si1)

        def clamp(t):
            return jnp.minimum(base + t * CHK, E - CHK)

        def start(t, s):
            cl = clamp(t)
            pltpu.make_async_copy(vals_ref.at[pl.ds(cl, CHK)], vb[s], sv[s]
                                  ).start()
            pltpu.make_async_copy(ridx_ref.at[pl.ds(cl, CHK)], ib[s], si[s]
                                  ).start()

        @pl.when(nch > 0)
        def _go():
            start(0, 0)

            def chunk_s(t, s):
                @pl.when(t + 1 < nch)
                def _pre():
                    start(t + 1, 1 - s)

                pltpu.make_async_copy(vals_ref.at[pl.ds(0, CHK)], vb[s], sv[s]
                                      ).wait()
                pltpu.make_async_copy(ridx_ref.at[pl.ds(0, CHK)], ib[s], si[s]
                                      ).wait()
                cur = base + t * CHK
                cl = clamp(t)
                p_row = cl + lax.broadcasted_iota(jnp.int32, (CHK, 1), 0)
                ok = (p_row >= jnp.maximum(cur, off0)) & (p_row < off1)
                lr = ib[s][...] - nstart
                hit = (lr == lax.broadcasted_iota(jnp.int32, (CHK, BN), 1)) & ok
                onehot = jnp.where(hit, 1.0, 0.0).astype(jnp.bfloat16)
                part = lax.dot_general(onehot, vb[s][...].astype(jnp.bfloat16),
                                       dimension_numbers=(((0,), (0,)), ((), ())),
                                       preferred_element_type=jnp.float32)
                o_ref[...] = o_ref[...] + part

            def pair(tt, c):
                @pl.when(2 * tt < nch)
                def _a():
                    chunk_s(2 * tt, 0)

                @pl.when(2 * tt + 1 < nch)
                def _b():
                    chunk_s(2 * tt + 1, 1)

                return c

            lax.fori_loop(0, lax.div(nch + 1, 2), pair, 0)

    return pl.pallas_call(
        body,
        grid=(NBLK,),
        in_specs=[
            pl.BlockSpec(memory_space=pltpu.MemorySpace.SMEM),
            pl.BlockSpec(memory_space=pltpu.MemorySpace.HBM),
            pl.BlockSpec(memory_space=pltpu.MemorySpace.HBM),
        ],
        out_specs=pl.BlockSpec((BN, H), lambda b: (b, 0)),
        out_shape=jax.ShapeDtypeStruct((N, H), jnp.float32),
        scratch_shapes=[
            pltpu.VMEM((CHK, H), jnp.float32),
            pltpu.VMEM((CHK, H), jnp.float32),
            pltpu.VMEM((CHK, 1), jnp.int32),
            pltpu.VMEM((CHK, 1), jnp.int32),
            pltpu.SemaphoreType.DMA,
            pltpu.SemaphoreType.DMA,
            pltpu.SemaphoreType.DMA,
            pltpu.SemaphoreType.DMA,
        ],
    )(offs, vals, ridx2d)


def _make_combine(relu):
    mesh = plsc.VectorSubcoreMesh(core_axis_name="c", subcore_axis_name="s",
                                  num_cores=NC, num_subcores=NS)

    scratch = [pltpu.VMEM((EPW,), jnp.int32), pltpu.VMEM((EPW,), jnp.int32)]
    for _ in range(2):
        scratch += [pltpu.VMEM((CBC, H), jnp.float32)] * 4
    scratch += [pltpu.SemaphoreType.DMA] * 8

    @functools.partial(
        pl.kernel,
        mesh=mesh,
        out_type=jax.ShapeDtypeStruct((E, H), jnp.float32),
        scratch_types=scratch,
    )
    def combine(ps_hbm, pr_hbm, e1_hbm, idxs_hbm, idxr_hbm, out_hbm, *bufs):
        idxs_v, idxr_v = bufs[0], bufs[1]
        ba = (bufs[2], bufs[6])
        bb = (bufs[3], bufs[7])
        bc = (bufs[4], bufs[8])
        bo = (bufs[5], bufs[9])
        sa = (bufs[10], bufs[11])
        sb = (bufs[12], bufs[13])
        sc = (bufs[14], bufs[15])
        so = (bufs[16], bufs[17])

        cid = lax.axis_index("c")
        sid = lax.axis_index("s")
        wid = sid * NC + cid
        g0 = wid * EPW
        pltpu.sync_copy(idxs_hbm.at[pl.ds(g0, EPW)], idxs_v)
        pltpu.sync_copy(idxr_hbm.at[pl.ds(g0, EPW)], idxr_v)

        def start_reads(j, s):
            pltpu.async_copy(ps_hbm.at[idxs_v.at[pl.ds(j * CBC, CBC)]],
                             ba[s], sa[s])
            pltpu.async_copy(pr_hbm.at[idxr_v.at[pl.ds(j * CBC, CBC)]],
                             bb[s], sb[s])
            pltpu.async_copy(e1_hbm.at[pl.ds(g0 + j * CBC, CBC)], bc[s], sc[s])

        def process(j, s):
            pltpu.make_async_copy(ps_hbm.at[pl.ds(0, CBC)], ba[s], sa[s]).wait()
            pltpu.make_async_copy(pr_hbm.at[pl.ds(0, CBC)], bb[s], sb[s]).wait()
            pltpu.make_async_copy(e1_hbm.at[pl.ds(0, CBC)], bc[s], sc[s]).wait()

            @pl.when(j >= 2)
            def _wait_prev_write():
                pltpu.make_async_copy(bo[s], out_hbm.at[pl.ds(0, CBC)],
                                      so[s]).wait()

            def row(r, rc):
                for kk in range(H // LANES):
                    sl = pl.ds(kk * LANES, LANES)
                    v = ba[s][r, sl] + bb[s][r, sl] + bc[s][r, sl]
                    if relu:
                        v = jnp.maximum(v, 0.0)
                    bo[s][r, sl] = v
                return rc

            lax.fori_loop(0, CBC, row, 0)
            pltpu.async_copy(bo[s], out_hbm.at[pl.ds(g0 + j * CBC, CBC)], so[s])

            @pl.when(j + 2 < NB)
            def _prefetch():
                start_reads(j + 2, s)

        start_reads(0, 0)
        start_reads(1, 1)

        def pair(jj, carry):
            process(2 * jj, 0)
            process(2 * jj + 1, 1)
            return carry

        lax.fori_loop(0, NB // 2, pair, 0)
        pltpu.make_async_copy(bo[0], out_hbm.at[pl.ds(0, CBC)], so[0]).wait()
        pltpu.make_async_copy(bo[1], out_hbm.at[pl.ds(0, CBC)], so[1]).wait()

    return combine


_combine_relu = _make_combine(True)
_combine_lin = _make_combine(False)


def kernel(node_features, senders, receivers, valid,
           W_e1, b_e1, W_n1, b_n1,
           W_e2, b_e2, W_n2, b_n2,
           W_e3, b_e3, W_n3, b_n3):
    perm = jnp.argsort(receivers)
    s_s = senders[perm]
    r_s = receivers[perm]
    valid_col = valid[perm].astype(jnp.float32).reshape(E, 1)
    offs = jnp.searchsorted(r_s, jnp.arange(0, N + 1, BN, dtype=jnp.int32)
                            ).astype(jnp.int32)
    r2d = r_s.reshape(E, 1)

    def esplit(We):
        K0 = We.shape[0] // 3
        return We[:K0], We[K0:2 * K0], We[2 * K0:]

    We1, Ws1, Wr1 = esplit(W_e1)
    We2, Ws2, Wr2 = esplit(W_e2)
    We3, Ws3, Wr3 = esplit(W_e3)

    def layer(nodes, Ps, Pr, e_prev, Wee, be, Wn, bn, relu, first,
              Wp1=None, Wp2=None):
        if first:
            e_term = _edge_bias(valid_col, jnp.sum(Wee, axis=0), be)
        else:
            e_term = _mm(e_prev, Wee, be)
        comb = _combine_relu if relu else _combine_lin
        e_new = comb(Ps, Pr, e_term, s_s, r_s)
        recv = _segsum(e_new, r2d, offs)
        Kn = nodes.shape[1]
        out = _mm_node(nodes, Wn[:Kn], recv, Wn[Kn:], bn, relu, Wp1, Wp2)
        return out, e_new

    Ps1, Pr1 = _mm2(node_features, Ws1, Wr1)
    (n1, Ps2, Pr2), e1 = layer(node_features, Ps1, Pr1, None, We1, b_e1,
                               W_n1, b_n1, True, True, Ws2, Wr2)
    (n2, Ps3, Pr3), e2 = layer(n1, Ps2, Pr2, e1, We2, b_e2,
                               W_n2, b_n2, True, False, Ws3, Wr3)
    n3, _ = layer(n2, Ps3, Pr3, e2, We3, b_e3, W_n3, b_n3, False, False)
    return n3


# speedup vs baseline: 2.4429x; 1.0765x over previous
import functools

import jax
import jax.numpy as jnp
from jax import lax
from jax.experimental import pallas as pl
from jax.experimental.pallas import tpu as pltpu
from jax.experimental.pallas import tpu_sc as plsc

N = 32768
E = 294912
D = 256
H = 256

NC = 2
NS = 16
LANES = 16
NW = NC * NS

CBC = 48
EPW = E // NW
NB = EPW // CBC

BN = 256
NBLK = N // BN
CHK = 1024


def _mm(A, W, b, relu=False, A2=None, W2=None, bm=1024):
    M, K = A.shape
    Nc = W.shape[1]

    if A2 is None:
        def body(a_ref, w_ref, b_ref, o_ref):
            acc = jnp.dot(a_ref[...], w_ref[...],
                          preferred_element_type=jnp.float32)
            acc = acc + b_ref[...]
            if relu:
                acc = jnp.maximum(acc, 0.0)
            o_ref[...] = acc
        in_specs = [
            pl.BlockSpec((bm, K), lambda m: (m, 0)),
            pl.BlockSpec((K, Nc), lambda m: (0, 0)),
            pl.BlockSpec((1, Nc), lambda m: (0, 0)),
        ]
        args = (A, W, b.reshape(1, Nc))
    else:
        K2 = A2.shape[1]

        def body(a_ref, w_ref, a2_ref, w2_ref, b_ref, o_ref):
            acc = jnp.dot(a_ref[...], w_ref[...],
                          preferred_element_type=jnp.float32)
            acc = acc + jnp.dot(a2_ref[...], w2_ref[...],
                                preferred_element_type=jnp.float32)
            acc = acc + b_ref[...]
            if relu:
                acc = jnp.maximum(acc, 0.0)
            o_ref[...] = acc
        in_specs = [
            pl.BlockSpec((bm, K), lambda m: (m, 0)),
            pl.BlockSpec((K, Nc), lambda m: (0, 0)),
            pl.BlockSpec((bm, K2), lambda m: (m, 0)),
            pl.BlockSpec((K2, Nc), lambda m: (0, 0)),
            pl.BlockSpec((1, Nc), lambda m: (0, 0)),
        ]
        args = (A, W, A2, W2, b.reshape(1, Nc))

    return pl.pallas_call(
        body,
        grid=(M // bm,),
        in_specs=in_specs,
        out_specs=pl.BlockSpec((bm, Nc), lambda m: (m, 0)),
        out_shape=jax.ShapeDtypeStruct((M, Nc), jnp.float32),
    )(*args)


def _mm_node(A, W1, recv, W2, b, relu, Wp1=None, Wp2=None, bm=1024):
    M, K = A.shape
    Nc = W1.shape[1]
    proj = Wp1 is not None

    def body(*refs):
        if proj:
            (a_ref, w1_ref, r_ref, w2_ref, b_ref, p1_ref, p2_ref,
             on_ref, o1_ref, o2_ref) = refs
        else:
            a_ref, w1_ref, r_ref, w2_ref, b_ref, on_ref = refs
        acc = jnp.dot(a_ref[...], w1_ref[...], preferred_element_type=jnp.float32)
        acc = acc + jnp.dot(r_ref[...], w2_ref[...],
                            preferred_element_type=jnp.float32)
        acc = acc + b_ref[...]
        if relu:
            acc = jnp.maximum(acc, 0.0)
        on_ref[...] = acc
        if proj:
            o1_ref[...] = jnp.dot(acc, p1_ref[...],
                                  preferred_element_type=jnp.float32)
            o2_ref[...] = jnp.dot(acc, p2_ref[...],
                                  preferred_element_type=jnp.float32)

    in_specs = [
        pl.BlockSpec((bm, K), lambda m: (m, 0)),
        pl.BlockSpec((K, Nc), lambda m: (0, 0)),
        pl.BlockSpec((bm, Nc), lambda m: (m, 0)),
        pl.BlockSpec((Nc, Nc), lambda m: (0, 0)),
        pl.BlockSpec((1, Nc), lambda m: (0, 0)),
    ]
    args = [A, W1, recv, W2, b.reshape(1, Nc)]
    out_specs = [pl.BlockSpec((bm, Nc), lambda m: (m, 0))]
    out_shape = [jax.ShapeDtypeStruct((M, Nc), jnp.float32)]
    if proj:
        in_specs += [pl.BlockSpec((Nc, Nc), lambda m: (0, 0))] * 2
        args += [Wp1, Wp2]
        out_specs += [pl.BlockSpec((bm, Nc), lambda m: (m, 0))] * 2
        out_shape += [jax.ShapeDtypeStruct((M, Nc), jnp.float32)] * 2

    res = pl.pallas_call(
        body,
        grid=(M // bm,),
        in_specs=in_specs,
        out_specs=out_specs,
        out_shape=out_shape,
    )(*args)
    return res if proj else res[0]


def _mm2(A, W1, W2, bm=1024):
    M, K = A.shape
    Nc = W1.shape[1]

    def body(a_ref, w1_ref, w2_ref, o1_ref, o2_ref):
        a = a_ref[...]
        o1_ref[...] = jnp.dot(a, w1_ref[...], preferred_element_type=jnp.float32)
        o2_ref[...] = jnp.dot(a, w2_ref[...], preferred_element_type=jnp.float32)

    return pl.pallas_call(
        body,
        grid=(M // bm,),
        in_specs=[
            pl.BlockSpec((bm, K), lambda m: (m, 0)),
            pl.BlockSpec((K, Nc), lambda m: (0, 0)),
            pl.BlockSpec((K, Nc), lambda m: (0, 0)),
        ],
        out_specs=[
            pl.BlockSpec((bm, Nc), lambda m: (m, 0)),
            pl.BlockSpec((bm, Nc), lambda m: (m, 0)),
        ],
        out_shape=[
            jax.ShapeDtypeStruct((M, Nc), jnp.float32),
            jax.ShapeDtypeStruct((M, Nc), jnp.float32),
        ],
    )(A, W1, W2)


def _edge_bias(valid_col, c1, be, bm=1024):
    def body(v_ref, c_ref, b_ref, o_ref):
        o_ref[...] = v_ref[...] * c_ref[...] + b_ref[...]

    return pl.pallas_call(
        body,
        grid=(E // bm,),
        in_specs=[
            pl.BlockSpec((bm, 1), lambda m: (m, 0)),
            pl.BlockSpec((1, H), lambda m: (0, 0)),
            pl.BlockSpec((1, H), lambda m: (0, 0)),
        ],
        out_specs=pl.BlockSpec((bm, H), lambda m: (m, 0)),
        out_shape=jax.ShapeDtypeStruct((E, H), jnp.float32),
    )(valid_col, c1.reshape(1, H), be.reshape(1, H))


def _segsum(vals, ridx2d, offs):
    def body(offs_ref, vals_ref, ridx_ref, o_ref,
             vbuf0, vbuf1, ibuf0, ibuf1, sv0, sv1, si0, si1):
        b = pl.program_id(0)
        off0 = offs_ref[b]
        off1 = offs_ref[b + 1]
        base = (off0 // 8) * 8
        nch = lax.div(off1 - base + (CHK - 1), CHK)
        nstart = b * BN
        o_ref[...] = jnp.zeros((BN, H), jnp.float32)
        vb = (vbuf0, vbuf1)
        ib = (ibuf0, ibuf1)
        sv = (sv0, sv1)
        si = (si0, si1)

        def clamp(t):
            return jnp.minimum(base + t * CHK, E - CHK)

        def start(t, s):
            cl = clamp(t)
            pltpu.make_async_copy(vals_ref.at[pl.ds(cl, CHK)], vb[s], sv[s]
                                  ).start()
            pltpu.make_async_copy(ridx_ref.at[pl.ds(cl, CHK)], ib[s], si[s]
                                  ).start()

        @pl.when(nch > 0)
        def _go():
            start(0, 0)

            def chunk_s(t, s):
                @pl.when(t + 1 < nch)
                def _pre():
                    start(t + 1, 1 - s)

                pltpu.make_async_copy(vals_ref.at[pl.ds(0, CHK)], vb[s], sv[s]
                                      ).wait()
                pltpu.make_async_copy(ridx_ref.at[pl.ds(0, CHK)], ib[s], si[s]
                                      ).wait()
                cur = base + t * CHK
                cl = clamp(t)
                p_row = cl + lax.broadcasted_iota(jnp.int32, (CHK, 1), 0)
                ok = (p_row >= jnp.maximum(cur, off0)) & (p_row < off1)
                lr = ib[s][...] - nstart
                hit = (lr == lax.broadcasted_iota(jnp.int32, (CHK, BN), 1)) & ok
                onehot = jnp.where(hit, 1.0, 0.0).astype(jnp.bfloat16)
                part = lax.dot_general(onehot, vb[s][...].astype(jnp.bfloat16),
                                       dimension_numbers=(((0,), (0,)), ((), ())),
                                       preferred_element_type=jnp.float32)
                o_ref[...] = o_ref[...] + part

            def pair(tt, c):
                @pl.when(2 * tt < nch)
                def _a():
                    chunk_s(2 * tt, 0)

                @pl.when(2 * tt + 1 < nch)
                def _b():
                    chunk_s(2 * tt + 1, 1)

                return c

            lax.fori_loop(0, lax.div(nch + 1, 2), pair, 0)

    return pl.pallas_call(
        body,
        grid=(NBLK,),
        in_specs=[
            pl.BlockSpec(memory_space=pltpu.MemorySpace.SMEM),
            pl.BlockSpec(memory_space=pltpu.MemorySpace.HBM),
            pl.BlockSpec(memory_space=pltpu.MemorySpace.HBM),
        ],
        out_specs=pl.BlockSpec((BN, H), lambda b: (b, 0)),
        out_shape=jax.ShapeDtypeStruct((N, H), jnp.float32),
        scratch_shapes=[
            pltpu.VMEM((CHK, H), jnp.float32),
            pltpu.VMEM((CHK, H), jnp.float32),
            pltpu.VMEM((CHK, 1), jnp.int32),
            pltpu.VMEM((CHK, 1), jnp.int32),
            pltpu.SemaphoreType.DMA,
            pltpu.SemaphoreType.DMA,
            pltpu.SemaphoreType.DMA,
            pltpu.SemaphoreType.DMA,
        ],
    )(offs, vals, ridx2d)


def _make_combine(relu):
    mesh = plsc.VectorSubcoreMesh(core_axis_name="c", subcore_axis_name="s",
                                  num_cores=NC, num_subcores=NS)

    scratch = [pltpu.VMEM((EPW,), jnp.int32), pltpu.VMEM((EPW,), jnp.int32)]
    for _ in range(2):
        scratch += [pltpu.VMEM((CBC, H), jnp.float32)] * 4
    scratch += [pltpu.SemaphoreType.DMA] * 8

    @functools.partial(
        pl.kernel,
        mesh=mesh,
        out_type=jax.ShapeDtypeStruct((E, H), jnp.float32),
        scratch_types=scratch,
    )
    def combine(ps_hbm, pr_hbm, e1_hbm, idxs_hbm, idxr_hbm, out_hbm, *bufs):
        idxs_v, idxr_v = bufs[0], bufs[1]
        ba = (bufs[2], bufs[6])
        bb = (bufs[3], bufs[7])
        bc = (bufs[4], bufs[8])
        bo = (bufs[5], bufs[9])
        sa = (bufs[10], bufs[11])
        sb = (bufs[12], bufs[13])
        sc = (bufs[14], bufs[15])
        so = (bufs[16], bufs[17])

        cid = lax.axis_index("c")
        sid = lax.axis_index("s")
        wid = sid * NC + cid
        g0 = wid * EPW
        pltpu.sync_copy(idxs_hbm.at[pl.ds(g0, EPW)], idxs_v)
        pltpu.sync_copy(idxr_hbm.at[pl.ds(g0, EPW)], idxr_v)

        def start_reads(j, s):
            pltpu.async_copy(ps_hbm.at[idxs_v.at[pl.ds(j * CBC, CBC)]],
                             ba[s], sa[s])
            pltpu.async_copy(pr_hbm.at[idxr_v.at[pl.ds(j * CBC, CBC)]],
                             bb[s], sb[s])
            pltpu.async_copy(e1_hbm.at[pl.ds(g0 + j * CBC, CBC)], bc[s], sc[s])

        def process(j, s):
            pltpu.make_async_copy(ps_hbm.at[pl.ds(0, CBC)], ba[s], sa[s]).wait()
            pltpu.make_async_copy(pr_hbm.at[pl.ds(0, CBC)], bb[s], sb[s]).wait()
            pltpu.make_async_copy(e1_hbm.at[pl.ds(0, CBC)], bc[s], sc[s]).wait()

            @pl.when(j >= 2)
            def _wait_prev_write():
                pltpu.make_async_copy(bo[s], out_hbm.at[pl.ds(0, CBC)],
                                      so[s]).wait()

            def row(r, rc):
                for kk in range(H // LANES):
                    sl = pl.ds(kk * LANES, LANES)
                    v = ba[s][r, sl] + bb[s][r, sl] + bc[s][r, sl]
                    if relu:
                        v = jnp.maximum(v, 0.0)
                    bo[s][r, sl] = v
                return rc

            lax.fori_loop(0, CBC, row, 0)
            pltpu.async_copy(bo[s], out_hbm.at[pl.ds(g0 + j * CBC, CBC)], so[s])

            @pl.when(j + 2 < NB)
            def _prefetch():
                start_reads(j + 2, s)

        start_reads(0, 0)
        start_reads(1, 1)

        def pair(jj, carry):
            process(2 * jj, 0)
            process(2 * jj + 1, 1)
            return carry

        lax.fori_loop(0, NB // 2, pair, 0)
        pltpu.make_async_copy(bo[0], out_hbm.at[pl.ds(0, CBC)], so[0]).wait()
        pltpu.make_async_copy(bo[1], out_hbm.at[pl.ds(0, CBC)], so[1]).wait()

    return combine


_combine_relu = _make_combine(True)
_combine_lin = _make_combine(False)


def kernel(node_features, senders, receivers, valid,
           W_e1, b_e1, W_n1, b_n1,
           W_e2, b_e2, W_n2, b_n2,
           W_e3, b_e3, W_n3, b_n3):
    perm = jnp.argsort(receivers)
    s_s = senders[perm]
    r_s = receivers[perm]
    valid_col = valid[perm].astype(jnp.float32).reshape(E, 1)
    offs = jnp.searchsorted(r_s, jnp.arange(0, N + 1, BN, dtype=jnp.int32)
                            ).astype(jnp.int32)
    r2d = r_s.reshape(E, 1)

    def esplit(We):
        K0 = We.shape[0] // 3
        return We[:K0], We[K0:2 * K0], We[2 * K0:]

    We1, Ws1, Wr1 = esplit(W_e1)
    We2, Ws2, Wr2 = esplit(W_e2)
    We3, Ws3, Wr3 = esplit(W_e3)

    def layer(nodes, Ps, Pr, e_prev, Wee, be, Wn, bn, relu, first,
              Wp1=None, Wp2=None):
        if first:
            e_term = _edge_bias(valid_col, jnp.sum(Wee, axis=0), be)
        else:
            e_term = _mm(e_prev, Wee, be)
        comb = _combine_relu if relu else _combine_lin
        e_new = comb(Ps, Pr, e_term, s_s, r_s)
        recv = _segsum(e_new, r2d, offs)
        Kn = nodes.shape[1]
        out = _mm_node(nodes, Wn[:Kn], recv, Wn[Kn:], bn, relu, Wp1, Wp2)
        return out, e_new

    Ps1, Pr1 = _mm2(node_features, Ws1, Wr1)
    (n1, Ps2, Pr2), e1 = layer(node_features, Ps1, Pr1, None, We1, b_e1,
                               W_n1, b_n1, True, True, Ws2, Wr2)
    (n2, Ps3, Pr3), e2 = layer(n1, Ps2, Pr2, e1, We2, b_e2,
                               W_n2, b_n2, True, False, Ws3, Wr3)
    n3, _ = layer(n2, Ps3, Pr3, e2, We3, b_e3, W_n3, b_n3, False, False)
    return n3
